# baseline (speedup 1.0000x reference)
import functools

import jax
import jax.numpy as jnp
from jax import lax
from jax.experimental import pallas as pl
from jax.experimental.pallas import tpu as pltpu
from jax.experimental.pallas import tpu_sc as plsc

N = 10000
D = 128
E = 320000
NC = 2
NS = 16
NW = NC * NS
B = 128
NB = -(-E // (NW * B))
E_PAD = NW * NB * B
N_PAD = 10240
TILE_ROWS = N_PAD // NS
R = 1024

_mesh = plsc.VectorSubcoreMesh(core_axis_name="c", subcore_axis_name="s")


@functools.partial(
    pl.kernel,
    out_type=jax.ShapeDtypeStruct((NC, N_PAD, D), jnp.float32),
    mesh=_mesh,
    scratch_types=[
        pltpu.VMEM((NB, B), jnp.int32),
        pltpu.VMEM((B, D), jnp.float32),
        pltpu.VMEM_SHARED((N_PAD, D), jnp.float32),
    ],
)
def _hist_sc(dstb, zeros, ones, out, dst_v, ones_v, hist_sh):
    c = lax.axis_index("c")
    s = lax.axis_index("s")
    wid = s * NC + c
    sl = pl.ds(s * TILE_ROWS, TILE_ROWS)
    pltpu.sync_copy(zeros.at[sl], hist_sh.at[sl])
    pltpu.sync_copy(dstb.at[wid], dst_v)
    pltpu.sync_copy(ones, ones_v)
    plsc.subcore_barrier()

    def body(b, carry):
        pltpu.sync_copy(ones_v, hist_sh.at[dst_v.at[b]], add=True)
        return carry

    lax.fori_loop(0, NB, body, 0)
    plsc.subcore_barrier()
    pltpu.sync_copy(hist_sh.at[sl], out.at[c, sl])


@functools.partial(
    pl.kernel,
    out_type=jax.ShapeDtypeStruct((NC, N_PAD, D), jnp.float32),
    mesh=_mesh,
    scratch_types=[
        pltpu.VMEM((NB, B), jnp.int32),
        pltpu.VMEM((NB, B), jnp.int32),
        pltpu.VMEM((B, D), jnp.float32),
        pltpu.SemaphoreType.DMA,
        pltpu.VMEM_SHARED((N_PAD, D), jnp.float32),
    ],
)
def _agg_sc(g, srcb, dstb, out, src_v, dst_v, rows_v, sem, acc_sh):
    c = lax.axis_index("c")
    s = lax.axis_index("s")
    wid = s * NC + c
    sl = pl.ds(s * TILE_ROWS, TILE_ROWS)
    pltpu.sync_copy(g.at[sl], acc_sh.at[sl])
    pltpu.sync_copy(srcb.at[wid], src_v)
    pltpu.sync_copy(dstb.at[wid], dst_v)
    plsc.subcore_barrier()

    def body(b, carry):
        pltpu.async_copy(g.at[src_v.at[b]], rows_v, sem).wait()
        pltpu.sync_copy(rows_v, acc_sh.at[dst_v.at[b]], add=True)
        return carry

    lax.fori_loop(0, NB, body, 0)
    plsc.subcore_barrier()
    pltpu.sync_copy(acc_sh.at[sl], out.at[c, sl])


def _dinv_from_hist(hist):
    deg = hist[0, :, 0:1] + hist[1, :, 0:1] + 1.0
    return lax.rsqrt(deg)


def _dense1_body(x_ref, w_ref, hist_ref, o_ref):
    h = lax.dot_general(
        x_ref[...], w_ref[...], (((1,), (0,)), ((), ())),
        preferred_element_type=jnp.float32, precision=lax.Precision.HIGHEST)
    o_ref[...] = h * _dinv_from_hist(hist_ref[...])


def _dense2_body(p_ref, g_ref, hist_ref, w_ref, b_ref, o_ref):
    dinv = _dinv_from_hist(hist_ref[...])
    z = dinv * (p_ref[0] + p_ref[1] - g_ref[...]) + b_ref[...]
    a = jnp.where(z >= 0, z, 0.01 * z)
    h = lax.dot_general(
        a, w_ref[...], (((1,), (0,)), ((), ())),
        preferred_element_type=jnp.float32, precision=lax.Precision.HIGHEST)
    o_ref[...] = h * dinv


def _dense3_body(p_ref, g_ref, hist_ref, b_ref, o_ref):
    dinv = _dinv_from_hist(hist_ref[...])
    z = dinv * (p_ref[0] + p_ref[1] - g_ref[...]) + b_ref[...]
    o_ref[...] = jnp.where(z >= 0, z, 0.01 * z)


_row_spec = pl.BlockSpec((R, D), lambda i: (i, 0))
_w_spec = pl.BlockSpec((D, D), lambda i: (0, 0))
_hist_spec = pl.BlockSpec((2, R, D), lambda i: (0, i, 0))
_p_spec = pl.BlockSpec((2, R, D), lambda i: (0, i, 0))
_b_spec = pl.BlockSpec((1, D), lambda i: (0, 0))
_GRID = (N_PAD // R,)
_row_out = jax.ShapeDtypeStruct((N_PAD, D), jnp.float32)

_dense1 = pl.pallas_call(
    _dense1_body, grid=_GRID,
    in_specs=[_row_spec, _w_spec, _hist_spec],
    out_specs=_row_spec, out_shape=_row_out)

_dense2 = pl.pallas_call(
    _dense2_body, grid=_GRID,
    in_specs=[_p_spec, _row_spec, _hist_spec, _w_spec, _b_spec],
    out_specs=_row_spec, out_shape=_row_out)

_dense3 = pl.pallas_call(
    _dense3_body, grid=_GRID,
    in_specs=[_p_spec, _row_spec, _hist_spec, _b_spec],
    out_specs=_row_spec, out_shape=_row_out)


def kernel(x, edge_index, W1, b1, W2, b2):
    src = edge_index[0].astype(jnp.int32)
    dst = edge_index[1].astype(jnp.int32)
    pad = N + (jnp.arange(E_PAD - E, dtype=jnp.int32) % (N_PAD - N))
    srcb = jnp.concatenate([src, pad]).reshape(NW, NB, B)
    dstb = jnp.concatenate([dst, pad]).reshape(NW, NB, B)
    x_pad = jnp.zeros((N_PAD, D), jnp.float32).at[:N].set(x)
    zeros = jnp.zeros((N_PAD, D), jnp.float32)
    ones = jnp.ones((B, D), jnp.float32)

    hist = _hist_sc(dstb, zeros, ones)
    g1 = _dense1(x_pad, W1, hist)
    p1 = _agg_sc(g1, srcb, dstb)
    g2 = _dense2(p1, g1, hist, W2, b1.reshape(1, D))
    p2 = _agg_sc(g2, srcb, dstb)
    out = _dense3(p2, g2, hist, b2.reshape(1, D))
    return out[:N]


# speedup vs baseline: 1.0045x; 1.0045x over previous
import functools

import jax
import jax.numpy as jnp
from jax import lax
from jax.experimental import pallas as pl
from jax.experimental.pallas import tpu as pltpu
from jax.experimental.pallas import tpu_sc as plsc

N = 10000
D = 128
E = 320000
NC = 2
NS = 16
NW = NC * NS
B = 128
NB = -(-E // (NW * B))
E_PAD = NW * NB * B
N_PAD = 10240
TILE_ROWS = N_PAD // NS
R = 1024

_mesh = plsc.VectorSubcoreMesh(core_axis_name="c", subcore_axis_name="s")


@functools.partial(
    pl.kernel,
    out_type=jax.ShapeDtypeStruct((NC, N_PAD, D), jnp.float32),
    mesh=_mesh,
    scratch_types=[
        pltpu.VMEM((NB, B), jnp.int32),
        pltpu.VMEM((B, D), jnp.float32),
        pltpu.VMEM_SHARED((N_PAD, D), jnp.float32),
    ],
)
def _hist_sc(dstb, zeros, ones, out, dst_v, ones_v, hist_sh):
    c = lax.axis_index("c")
    s = lax.axis_index("s")
    wid = s * NC + c
    sl = pl.ds(s * TILE_ROWS, TILE_ROWS)
    pltpu.sync_copy(zeros.at[sl], hist_sh.at[sl])
    pltpu.sync_copy(dstb.at[wid], dst_v)
    pltpu.sync_copy(ones, ones_v)
    plsc.subcore_barrier()

    def body(b, carry):
        pltpu.sync_copy(ones_v, hist_sh.at[dst_v.at[b]], add=True)
        return carry

    lax.fori_loop(0, NB, body, 0)
    plsc.subcore_barrier()
    pltpu.sync_copy(hist_sh.at[sl], out.at[c, sl])


@functools.partial(
    pl.kernel,
    out_type=jax.ShapeDtypeStruct((NC, N_PAD, D), jnp.float32),
    mesh=_mesh,
    scratch_types=[
        pltpu.VMEM((NB, B), jnp.int32),
        pltpu.VMEM((NB, B), jnp.int32),
        pltpu.VMEM((B, D), jnp.float32),
        pltpu.SemaphoreType.DMA,
        pltpu.VMEM_SHARED((N_PAD, D), jnp.float32),
    ],
)
def _agg_sc(g, srcb, dstb, out, src_v, dst_v, rows_v, sem, acc_sh):
    c = lax.axis_index("c")
    s = lax.axis_index("s")
    wid = s * NC + c
    sl = pl.ds(s * TILE_ROWS, TILE_ROWS)
    pltpu.sync_copy(g.at[sl], acc_sh.at[sl])
    pltpu.sync_copy(srcb.at[wid], src_v)
    pltpu.sync_copy(dstb.at[wid], dst_v)
    plsc.subcore_barrier()

    def body(b, carry):
        pltpu.async_copy(g.at[src_v.at[b]], rows_v, sem).wait()
        pltpu.sync_copy(rows_v, acc_sh.at[dst_v.at[b]], add=True)
        return carry

    lax.fori_loop(0, NB, body, 0)
    plsc.subcore_barrier()
    pltpu.sync_copy(acc_sh.at[sl], out.at[c, sl])


def _dinv_from_hist(hist):
    deg = hist[0, :, 0:1] + hist[1, :, 0:1] + 1.0
    return lax.rsqrt(deg)


def _dense1_body(x_ref, w_ref, hist_ref, o_ref, dinv_ref):
    dinv = _dinv_from_hist(hist_ref[...])
    h = lax.dot_general(
        x_ref[...], w_ref[...], (((1,), (0,)), ((), ())),
        preferred_element_type=jnp.float32, precision=lax.Precision.HIGHEST)
    o_ref[...] = h * dinv
    dinv_ref[...] = jnp.broadcast_to(dinv, (R, D))


def _dense2_body(p_ref, g_ref, dinvr_ref, w_ref, b_ref, o_ref):
    dinv = dinvr_ref[...]
    z = dinv * (p_ref[0] + p_ref[1] - g_ref[...]) + b_ref[...]
    a = jnp.where(z >= 0, z, 0.01 * z)
    h = lax.dot_general(
        a, w_ref[...], (((1,), (0,)), ((), ())),
        preferred_element_type=jnp.float32, precision=lax.Precision.HIGHEST)
    o_ref[...] = h * dinv


def _dense3_body(p_ref, g_ref, dinvr_ref, b_ref, o_ref):
    dinv = dinvr_ref[...]
    z = dinv * (p_ref[0] + p_ref[1] - g_ref[...]) + b_ref[...]
    o_ref[...] = jnp.where(z >= 0, z, 0.01 * z)


_row_spec = pl.BlockSpec((R, D), lambda i: (i, 0))
_w_spec = pl.BlockSpec((D, D), lambda i: (0, 0))
_hist_spec = pl.BlockSpec((2, R, D), lambda i: (0, i, 0))
_p_spec = pl.BlockSpec((2, R, D), lambda i: (0, i, 0))
_b_spec = pl.BlockSpec((1, D), lambda i: (0, 0))
_GRID = (N_PAD // R,)
_row_out = jax.ShapeDtypeStruct((N_PAD, D), jnp.float32)

_dense1 = pl.pallas_call(
    _dense1_body, grid=_GRID,
    in_specs=[_row_spec, _w_spec, _hist_spec],
    out_specs=[_row_spec, _row_spec], out_shape=[_row_out, _row_out])

_dense2 = pl.pallas_call(
    _dense2_body, grid=_GRID,
    in_specs=[_p_spec, _row_spec, _row_spec, _w_spec, _b_spec],
    out_specs=_row_spec, out_shape=_row_out)

_dense3 = pl.pallas_call(
    _dense3_body, grid=_GRID,
    in_specs=[_p_spec, _row_spec, _row_spec, _b_spec],
    out_specs=_row_spec, out_shape=_row_out)


def kernel(x, edge_index, W1, b1, W2, b2):
    src = edge_index[0].astype(jnp.int32)
    dst = edge_index[1].astype(jnp.int32)
    pad = N + (jnp.arange(E_PAD - E, dtype=jnp.int32) % (N_PAD - N))
    srcb = jnp.concatenate([src, pad]).reshape(NW, NB, B)
    dstb = jnp.concatenate([dst, pad]).reshape(NW, NB, B)
    x_pad = jnp.zeros((N_PAD, D), jnp.float32).at[:N].set(x)
    zeros = jnp.zeros((N_PAD, D), jnp.float32)
    ones = jnp.ones((B, D), jnp.float32)

    hist = _hist_sc(dstb, zeros, ones)
    g1, dinvr = _dense1(x_pad, W1, hist)
    p1 = _agg_sc(g1, srcb, dstb)
    g2 = _dense2(p1, g1, dinvr, W2, b1.reshape(1, D))
    p2 = _agg_sc(g2, srcb, dstb)
    out = _dense3(p2, g2, dinvr, b2.reshape(1, D))
    return out[:N]
